# baseline (device time: 145663 ns/iter reference)
import os

import jax
import jax.numpy as jnp
from jax import lax
from jax.experimental import pallas as pl
from jax.experimental.pallas import tpu as pltpu

N_DEV = 4

_PROBE = os.environ.get("PROBE_GEMM", "")


def _kernel_probe(x, w_mat, scale_x, scale_w):
    m_per, k = x.shape
    _, n_per = w_mat.shape
    xq = x.astype(jnp.float8_e4m3fn)
    wq = w_mat.astype(jnp.float8_e5m2)

    def body(x_ref, w_ref, sx_ref, sw_ref, out_ref):
        scale = sx_ref[0] * sw_ref[0]
        for h in range(N_DEV):
            acc = lax.dot_general(
                x_ref[...], w_ref[...],
                (((1,), (0,)), ((), ())),
                preferred_element_type=jnp.float32,
            )
            y = acc * scale
            out_ref[pl.ds(h * m_per, m_per), :] = y * jax.nn.sigmoid(y)

    return pl.pallas_call(
        body,
        out_shape=jax.ShapeDtypeStruct((N_DEV * m_per, n_per), jnp.float32),
        in_specs=[
            pl.BlockSpec(memory_space=pltpu.VMEM),
            pl.BlockSpec(memory_space=pltpu.VMEM),
            pl.BlockSpec(memory_space=pltpu.SMEM),
            pl.BlockSpec(memory_space=pltpu.SMEM),
        ],
        out_specs=pl.BlockSpec(memory_space=pltpu.VMEM),
        compiler_params=pltpu.CompilerParams(
            vmem_limit_bytes=100 * 1024 * 1024,
        ),
    )(xq, wq, scale_x, scale_w)


def _kernel_real(x, w_mat, scale_x, scale_w):
    m_per, k = x.shape
    k2, n_per = w_mat.shape
    assert k2 == k
    half = m_per // 2

    xq = x.astype(jnp.float8_e4m3fn)
    wq = w_mat.astype(jnp.float8_e5m2)

    def body(x_ref, w_ref, sx_ref, sw_ref, out_ref,
             bufL, bufR, bufO, send_sems, recv_sems):
        my = lax.axis_index("i")
        left = lax.rem(my + (N_DEV - 1), N_DEV)
        right = lax.rem(my + 1, N_DEV)

        barrier_sem = pltpu.get_barrier_semaphore()
        for nbr in (left, right):
            pl.semaphore_signal(
                barrier_sem, inc=1,
                device_id=(nbr,), device_id_type=pl.DeviceIdType.MESH,
            )
        pl.semaphore_wait(barrier_sem, 2)

        scale = sx_ref[0] * sw_ref[0]

        def gemm_store(chunk, origin):
            acc = lax.dot_general(
                chunk, w_ref[...],
                (((1,), (0,)), ((), ())),
                preferred_element_type=jnp.float32,
            )
            y = acc * scale
            out_ref[pl.ds(origin * m_per, m_per), :] = y * jax.nn.sigmoid(y)

        def rcopy(src, dst, i, dev):
            return pltpu.make_async_remote_copy(
                src_ref=src, dst_ref=dst,
                send_sem=send_sems.at[i], recv_sem=recv_sems.at[i],
                device_id=(dev,), device_id_type=pl.DeviceIdType.MESH,
            )

        r1 = rcopy(x_ref, bufL, 0, right)
        r2 = rcopy(x_ref, bufR, 1, left)
        r1.start()
        r2.start()
        gemm_store(x_ref[...], my)
        r1.wait_recv()
        r3 = rcopy(bufL.at[pl.ds(0, half)], bufO.at[pl.ds(0, half)], 2, right)
        r3.start()
        r2.wait_recv()
        r4 = rcopy(bufR.at[pl.ds(half, half)], bufO.at[pl.ds(half, half)], 3, left)
        r4.start()
        gemm_store(bufL[...], lax.rem(my + (N_DEV - 1), N_DEV))
        gemm_store(bufR[...], lax.rem(my + 1, N_DEV))
        r3.wait_recv()
        r4.wait_recv()
        gemm_store(bufO[...], lax.rem(my + 2, N_DEV))
        for r in (r1, r2, r3, r4):
            r.wait_send()

    return pl.pallas_call(
        body,
        out_shape=jax.ShapeDtypeStruct((N_DEV * m_per, n_per), jnp.float32),
        in_specs=[
            pl.BlockSpec(memory_space=pltpu.VMEM),
            pl.BlockSpec(memory_space=pltpu.VMEM),
            pl.BlockSpec(memory_space=pltpu.SMEM),
            pl.BlockSpec(memory_space=pltpu.SMEM),
        ],
        out_specs=pl.BlockSpec(memory_space=pltpu.VMEM),
        scratch_shapes=[
            pltpu.VMEM((m_per, k), jnp.float8_e4m3fn),
            pltpu.VMEM((m_per, k), jnp.float8_e4m3fn),
            pltpu.VMEM((m_per, k), jnp.float8_e4m3fn),
            pltpu.SemaphoreType.DMA((4,)),
            pltpu.SemaphoreType.DMA((4,)),
        ],
        compiler_params=pltpu.CompilerParams(
            collective_id=0,
            vmem_limit_bytes=100 * 1024 * 1024,
        ),
    )(xq, wq, scale_x, scale_w)


kernel = _kernel_probe if _PROBE else _kernel_real


# device time: 128958 ns/iter; 1.1295x vs baseline; 1.1295x over previous
import os

import jax
import jax.numpy as jnp
from jax import lax
from jax.experimental import pallas as pl
from jax.experimental.pallas import tpu as pltpu

N_DEV = 4

_PROBE = os.environ.get("PROBE_GEMM", "")


def _kernel_probe(x, w_mat, scale_x, scale_w):
    m_per, k = x.shape
    _, n_per = w_mat.shape
    xq = x.astype(jnp.float8_e4m3fn)
    wq = w_mat.astype(jnp.float8_e5m2)

    def body(x_ref, w_ref, sx_ref, sw_ref, out_ref):
        scale = sx_ref[0] * sw_ref[0]
        for h in range(N_DEV):
            acc = lax.dot_general(
                x_ref[...], w_ref[...],
                (((1,), (0,)), ((), ())),
                preferred_element_type=jnp.float32,
            )
            y = acc * scale
            out_ref[pl.ds(h * m_per, m_per), :] = y * jax.nn.sigmoid(y)

    return pl.pallas_call(
        body,
        out_shape=jax.ShapeDtypeStruct((N_DEV * m_per, n_per), jnp.float32),
        in_specs=[
            pl.BlockSpec(memory_space=pltpu.VMEM),
            pl.BlockSpec(memory_space=pltpu.VMEM),
            pl.BlockSpec(memory_space=pltpu.SMEM),
            pl.BlockSpec(memory_space=pltpu.SMEM),
        ],
        out_specs=pl.BlockSpec(memory_space=pltpu.VMEM),
        compiler_params=pltpu.CompilerParams(
            vmem_limit_bytes=100 * 1024 * 1024,
        ),
    )(xq, wq, scale_x, scale_w)


def _kernel_real(x, w_mat, scale_x, scale_w):
    m_per, k = x.shape
    k2, n_per = w_mat.shape
    assert k2 == k
    half = m_per // 2
    KT = 16
    kt = k // KT

    xq_in = x.astype(jnp.float8_e4m3fn)

    def body(x_ref, w_ref, sx_ref, sw_ref, out_ref,
             wstage, wq, bufL, bufR, bufO, send_sems, recv_sems, wsem):
        my = lax.axis_index("i")
        left = lax.rem(my + (N_DEV - 1), N_DEV)
        right = lax.rem(my + 1, N_DEV)

        barrier_sem = pltpu.get_barrier_semaphore()
        for nbr in (left, right):
            pl.semaphore_signal(
                barrier_sem, inc=1,
                device_id=(nbr,), device_id_type=pl.DeviceIdType.MESH,
            )
        pl.semaphore_wait(barrier_sem, 2)

        scale = sx_ref[0] * sw_ref[0]

        def gemm_store(chunk, origin):
            acc = lax.dot_general(
                chunk, wq[...],
                (((1,), (0,)), ((), ())),
                preferred_element_type=jnp.float32,
            )
            y = acc * scale
            out_ref[pl.ds(origin * m_per, m_per), :] = y * jax.nn.sigmoid(y)

        def rcopy(src, dst, i, dev):
            return pltpu.make_async_remote_copy(
                src_ref=src, dst_ref=dst,
                send_sem=send_sems.at[i], recv_sem=recv_sems.at[i],
                device_id=(dev,), device_id_type=pl.DeviceIdType.MESH,
            )

        r1 = rcopy(x_ref, bufL, 0, right)
        r2 = rcopy(x_ref, bufR, 1, left)
        r1.start()
        r2.start()
        def wcopy(t, slot):
            return pltpu.make_async_copy(
                w_ref.at[pl.ds(t * kt, kt)], wstage.at[slot], wsem.at[slot],
            )
        wcopy(0, 0).start()
        for t in range(KT):
            if t + 1 < KT:
                wcopy(t + 1, (t + 1) % 2).start()
            wcopy(t, t % 2).wait()
            wq[pl.ds(t * kt, kt), :] = wstage[t % 2].astype(jnp.float8_e5m2)
        gemm_store(x_ref[...], my)
        r1.wait_recv()
        r3 = rcopy(bufL.at[pl.ds(0, half)], bufO.at[pl.ds(0, half)], 2, right)
        r3.start()
        r2.wait_recv()
        r4 = rcopy(bufR.at[pl.ds(half, half)], bufO.at[pl.ds(half, half)], 3, left)
        r4.start()
        gemm_store(bufL[...], lax.rem(my + (N_DEV - 1), N_DEV))
        gemm_store(bufR[...], lax.rem(my + 1, N_DEV))
        r3.wait_recv()
        r4.wait_recv()
        gemm_store(bufO[...], lax.rem(my + 2, N_DEV))
        for r in (r1, r2, r3, r4):
            r.wait_send()

    return pl.pallas_call(
        body,
        out_shape=jax.ShapeDtypeStruct((N_DEV * m_per, n_per), jnp.float32),
        in_specs=[
            pl.BlockSpec(memory_space=pltpu.VMEM),
            pl.BlockSpec(memory_space=pl.ANY),
            pl.BlockSpec(memory_space=pltpu.SMEM),
            pl.BlockSpec(memory_space=pltpu.SMEM),
        ],
        out_specs=pl.BlockSpec(memory_space=pltpu.VMEM),
        scratch_shapes=[
            pltpu.VMEM((2, kt, n_per), jnp.float32),
            pltpu.VMEM((k, n_per), jnp.float8_e5m2),
            pltpu.VMEM((m_per, k), jnp.float8_e4m3fn),
            pltpu.VMEM((m_per, k), jnp.float8_e4m3fn),
            pltpu.VMEM((m_per, k), jnp.float8_e4m3fn),
            pltpu.SemaphoreType.DMA((4,)),
            pltpu.SemaphoreType.DMA((4,)),
            pltpu.SemaphoreType.DMA((2,)),
        ],
        compiler_params=pltpu.CompilerParams(
            collective_id=0,
            vmem_limit_bytes=63 * 1024 * 1024,
        ),
    )(xq_in, w_mat, scale_x, scale_w)


kernel = _kernel_probe if _PROBE else _kernel_real
